# baseline (speedup 1.0000x reference)
import jax
import jax.numpy as jnp
from jax.experimental import pallas as pl

POS_DIM = 128


def _gcn_norm(row, col, num_nodes):
    loop = jnp.arange(num_nodes, dtype=row.dtype)
    row = jnp.concatenate([row, loop])
    col = jnp.concatenate([col, loop])
    ones = jnp.ones(row.shape[0], dtype=jnp.float32)
    deg = jnp.zeros(num_nodes, dtype=jnp.float32).at[col].add(ones)
    dis = jnp.where(deg > 0, jax.lax.rsqrt(deg), 0.0)
    ew = dis[row] * dis[col]
    return row, col, ew


def _peg_conv(x, edge_index, ew1, eb1, ew2, eb2, lw, lb):
    coors = x[:, :POS_DIM]
    feats = x[:, POS_DIM:]
    n = x.shape[0]
    row, col, enorm = _gcn_norm(edge_index[0], edge_index[1], n)
    rel = coors[row] - coors[col]
    rel_dist = jnp.sum(rel * rel, axis=-1, keepdims=True)
    h = rel_dist @ ew1 + eb1
    h = h @ ew2 + eb2
    pe_w = jax.nn.sigmoid(h)
    msg = pe_w * enorm[:, None] * feats[row]
    aggr = jnp.zeros((n, feats.shape[1]), dtype=feats.dtype).at[col].add(msg)
    out = aggr @ lw + lb
    return jnp.concatenate([coors, out], axis=-1)


def _affine_body(p_ref, pe_ref, w_ref, out_ref):
    out_ref[...] = (p_ref[...] * w_ref[0, 0] + pe_ref[...] * w_ref[1, 0]
                    + w_ref[2, 0])


def kernel(x, edge_index, idx, c1_ew1, c1_eb1, c1_ew2, c1_eb2, c1_lw, c1_lb,
           c2_ew1, c2_eb1, c2_ew2, c2_eb2, c2_lw, c2_lb, fc_w, fc_b):
    h = _peg_conv(x, edge_index, c1_ew1, c1_eb1, c1_ew2, c1_eb2, c1_lw, c1_lb)
    h = _peg_conv(h, edge_index, c2_ew1, c2_eb1, c2_ew2, c2_eb2, c2_lw, c2_lb)
    pos = h[:, :POS_DIM]
    feats = h[:, POS_DIM:]
    nodes_first = feats[idx[0]]
    nodes_second = feats[idx[1]]
    pos_first = pos[idx[0]]
    pos_second = pos[idx[1]]
    pe = jnp.sum((pos_first - pos_second) ** 2, axis=-1, keepdims=True)
    pred = jnp.sum(nodes_first * nodes_second, axis=-1, keepdims=True)
    w = jnp.concatenate([fc_w, fc_b[None, :]], axis=0)
    p2 = pred.reshape(512, 128)
    pe2 = pe.reshape(512, 128)
    out = pl.pallas_call(
        _affine_body,
        out_shape=jax.ShapeDtypeStruct(p2.shape, p2.dtype),
    )(p2, pe2, w)
    return out.reshape(-1, 1)


# speedup vs baseline: 1.4294x; 1.4294x over previous
import functools

import jax
import jax.numpy as jnp
from jax import lax
from jax.experimental import pallas as pl
from jax.experimental.pallas import tpu as pltpu
from jax.experimental.pallas import tpu_sc as plsc

N = 10000
PD = 128
F = 256
E = 160000
EPAD = 163840
NW = 32
EPW = EPAD // NW
EPT = EPAD // 16
NPAIR = 65536
PPW = NPAIR // NW
CH = 128
NCH1 = EPW // CH
NCH3 = EPT // CH
NCH5 = PPW // CH
NPAD = 10240
NROW_T = NPAD // 16


def _mesh():
    return plsc.VectorSubcoreMesh(core_axis_name="c", subcore_axis_name="s")


_SC_PARAMS = pltpu.CompilerParams(needs_layout_passes=False)


def _wid():
    return lax.axis_index("c") * 16 + lax.axis_index("s")


def _iota16():
    return lax.broadcasted_iota(jnp.int32, (16,), 0)


def _deg_body(col_hbm, deg_out, col_v, deg_v):
    wid = _wid()
    base = wid * EPW
    pltpu.sync_copy(col_hbm.at[pl.ds(base, EPW)], col_v)

    lane = _iota16()

    def zero2(i, _):
        plsc.store_scatter(deg_v, [jnp.full((16,), i, jnp.int32), lane],
                           jnp.zeros((16,), jnp.float32))
        return 0
    lax.fori_loop(0, N // 16, zero2, 0)

    def upd(i, _):
        cv = col_v[pl.ds(i * 16, 16)]
        gid = base + i * 16 + lane
        val = jnp.where(gid < E, jnp.float32(1.0), jnp.float32(0.0))
        plsc.addupdate_scatter(deg_v, [cv >> 4, cv & 15], val)
        return 0
    lax.fori_loop(0, EPW // 16, upd, 0)
    pltpu.sync_copy(deg_v, deg_out.at[wid])


def _deg_call(col_pad):
    f = pl.kernel(
        _deg_body,
        out_type=jax.ShapeDtypeStruct((NW, N // 16, 16), jnp.float32),
        mesh=_mesh(),
        compiler_params=_SC_PARAMS,
        scratch_types=[
            pltpu.VMEM((EPW,), jnp.int32),
            pltpu.VMEM((N // 16, 16), jnp.float32),
        ],
    )
    return f(col_pad)


def _prep_body(x_ref, dp_ref, e1w1, e1b1, e1w2, e1b2, e2w1, e2b1, e2w2, e2b2,
               fcw, fcb, coors_ref, ff_ref, dis_ref, s1_ref, s2_ref, pv_ref):
    xv = x_ref[...]
    coors_ref[...] = xv[:, :PD]
    ff_ref[0:N, :] = xv[:, PD:PD + 128]
    ff_ref[NPAD:NPAD + N, :] = xv[:, PD + 128:]
    deg = jnp.sum(dp_ref[...], axis=0, keepdims=True) + 1.0
    dis_ref[...] = lax.rsqrt(deg)
    a1 = jnp.sum(e1w1[...] * e1w2[...])
    c1 = jnp.sum(e1b1[...] * e1w2[...]) + e1b2[0, 0]
    a2 = jnp.sum(e2w1[...] * e2w2[...])
    c2 = jnp.sum(e2b1[...] * e2w2[...]) + e2b2[0, 0]
    s1_ref[...] = jax.nn.sigmoid(c1) / deg
    s2_ref[...] = jax.nn.sigmoid(c2) / deg
    k = lax.broadcasted_iota(jnp.int32, (1, 16), 1)
    pv = jnp.where(k == 0, a1, 0.0)
    pv = jnp.where(k == 1, c1, pv)
    pv = jnp.where(k == 2, a2, pv)
    pv = jnp.where(k == 3, c2, pv)
    pv = jnp.where(k == 4, fcw[0, 0], pv)
    pv = jnp.where(k == 5, fcw[0, 1], pv)
    pv = jnp.where(k == 6, fcb[0, 0], pv)
    pv_ref[...] = pv


def _prep_call(x, dp, e1w1, e1b1, e1w2, e1b2, e2w1, e2b1, e2w2, e2b2, fcw, fcb):
    return pl.pallas_call(
        _prep_body,
        out_shape=[
            jax.ShapeDtypeStruct((N, PD), jnp.float32),
            jax.ShapeDtypeStruct((2 * NPAD, 128), jnp.float32),
            jax.ShapeDtypeStruct((1, N), jnp.float32),
            jax.ShapeDtypeStruct((1, N), jnp.float32),
            jax.ShapeDtypeStruct((1, N), jnp.float32),
            jax.ShapeDtypeStruct((1, 16), jnp.float32),
        ],
    )(x, dp, e1w1, e1b1, e1w2, e1b2, e2w1, e2b1, e2w2, e2b2, fcw, fcb)


def _coef_body(row_hbm, col_hbm, coors_hbm, dis_hbm, pv_hbm,
               coef1_hbm, coef2_hbm,
               row_v, col_v, dis_v, pv_v, cR, cC, c1_v, c2_v, sem):
    wid = _wid()
    base = wid * EPW
    pltpu.sync_copy(dis_hbm, dis_v)
    pltpu.sync_copy(pv_hbm, pv_v)
    pltpu.sync_copy(row_hbm.at[pl.ds(base, EPW)], row_v)
    pltpu.sync_copy(col_hbm.at[pl.ds(base, EPW)], col_v)
    pvv = pv_v[...]
    a1 = pvv[0]
    cc1 = pvv[1]
    a2 = pvv[2]
    cc2 = pvv[3]
    lane = _iota16()

    def chunk(j, _):
        cpR = pltpu.async_copy(coors_hbm.at[row_v.at[pl.ds(j * CH, CH)]], cR, sem)
        cpC = pltpu.async_copy(coors_hbm.at[col_v.at[pl.ds(j * CH, CH)]], cC, sem)
        cpR.wait()
        cpC.wait()

        def grp(v, _):
            eids = v * 16 + lane
            rel = jnp.zeros((16,), jnp.float32)
            for d in range(PD):
                dv = jnp.full((16,), d, jnp.int32)
                df = plsc.load_gather(cR, [eids, dv]) - plsc.load_gather(cC, [eids, dv])
                rel = rel + df * df
            rlo = row_v[pl.ds(j * CH + v * 16, 16)]
            clo = col_v[pl.ds(j * CH + v * 16, 16)]
            enorm = (plsc.load_gather(dis_v, [rlo >> 4, rlo & 15])
                     * plsc.load_gather(dis_v, [clo >> 4, clo & 15]))
            gid = base + j * CH + v * 16 + lane
            valid = gid < E
            w1 = 1.0 / (1.0 + jnp.exp(-(a1 * rel + cc1)))
            w2 = 1.0 / (1.0 + jnp.exp(-(a2 * rel + cc2)))
            c1_v[pl.ds(v * 16, 16)] = jnp.where(valid, enorm * w1, 0.0)
            c2_v[pl.ds(v * 16, 16)] = jnp.where(valid, enorm * w2, 0.0)
            return 0
        lax.fori_loop(0, CH // 16, grp, 0)
        pltpu.sync_copy(c1_v, coef1_hbm.at[pl.ds(base + j * CH, CH)])
        pltpu.sync_copy(c2_v, coef2_hbm.at[pl.ds(base + j * CH, CH)])
        return 0
    lax.fori_loop(0, NCH1, chunk, 0)


def _coef_call(row_pad, col_pad, coors, dis, pv):
    f = pl.kernel(
        _coef_body,
        out_type=(
            jax.ShapeDtypeStruct((EPAD,), jnp.float32),
            jax.ShapeDtypeStruct((EPAD,), jnp.float32),
        ),
        mesh=_mesh(),
        compiler_params=_SC_PARAMS,
        scratch_types=[
            pltpu.VMEM((EPW,), jnp.int32),
            pltpu.VMEM((EPW,), jnp.int32),
            pltpu.VMEM((N // 16, 16), jnp.float32),
            pltpu.VMEM((16,), jnp.float32),
            pltpu.VMEM((CH, PD), jnp.float32),
            pltpu.VMEM((CH, PD), jnp.float32),
            pltpu.VMEM((CH,), jnp.float32),
            pltpu.VMEM((CH,), jnp.float32),
            pltpu.SemaphoreType.DMA,
        ],
    )
    return f(row_pad, col_pad, coors, dis, pv)


ACC = 6144


def _spmm_body(row_hbm, col_hbm, coef_hbm, ff_hbm, aggr_hbm,
               row_v, col_v, colp_v, coef_v, msg_v, zero_v, agg_sh, sem):
    cidx = lax.axis_index("c")
    sidx = lax.axis_index("s")
    tbase = sidx * EPT
    pltpu.sync_copy(row_hbm.at[pl.ds(tbase, EPT)], row_v)
    pltpu.sync_copy(col_hbm.at[pl.ds(tbase, EPT)], col_v)
    pltpu.sync_copy(coef_hbm.at[pl.ds(tbase, EPT)], coef_v)

    off = cidx * NPAD

    def adj(i, _):
        row_v[pl.ds(i * 16, 16)] = row_v[pl.ds(i * 16, 16)] + off
        return 0
    lax.fori_loop(0, EPT // 16, adj, 0)

    def zbuf(i, _):
        zero_v[i // 8, pl.ds((i % 8) * 16, 16)] = jnp.zeros((16,), jnp.float32)
        return 0
    lax.fori_loop(0, (128 * 128) // 16, zbuf, 0)

    for pp in range(2):
        pbase = pp * ACC
        prows = ACC if pp == 0 else NPAD - ACC
        rows_t = prows // 16

        def redir(i, _):
            r = i // 8
            v = i % 8
            cv = col_v[pl.ds(i * 16, 16)]
            lv = cv - pbase
            ok = (lv >= 0) & (lv < prows)
            colp_v[r, pl.ds(v * 16, 16)] = jnp.where(ok, lv, ACC)
            return 0
        lax.fori_loop(0, EPT // 16, redir, 0)

        for k in range(rows_t // 128):
            pltpu.sync_copy(zero_v,
                            agg_sh.at[pl.ds(sidx * rows_t + k * 128, 128)])
        plsc.subcore_barrier()

        def chunk(j, _):
            pltpu.async_copy(ff_hbm.at[row_v.at[pl.ds(j * CH, CH)]], msg_v,
                             sem).wait()

            def scale(e, _):
                csv = plsc.load_gather(
                    coef_v, [jnp.full((16,), j * CH + e, jnp.int32)])
                ev = jnp.full((16,), e, jnp.int32)
                for v in range(8):
                    dsub = v * 16 + _iota16()
                    m = plsc.load_gather(msg_v, [ev, dsub])
                    plsc.store_scatter(msg_v, [ev, dsub], m * csv)
                return 0
            lax.fori_loop(0, CH, scale, 0)
            pltpu.sync_copy(msg_v, agg_sh.at[colp_v.at[j]], add=True)
            return 0
        lax.fori_loop(0, NCH3, chunk, 0)
        plsc.subcore_barrier()
        pltpu.sync_copy(
            agg_sh.at[pl.ds(sidx * rows_t, rows_t)],
            aggr_hbm.at[pl.ds(cidx * NPAD + pbase + sidx * rows_t, rows_t)])
        if pp == 0:
            plsc.subcore_barrier()


def _spmm_call(row_pad, col_pad, coef, ff):
    f = pl.kernel(
        _spmm_body,
        out_type=jax.ShapeDtypeStruct((2 * NPAD, 128), jnp.float32),
        mesh=_mesh(),
        compiler_params=_SC_PARAMS,
        scratch_types=[
            pltpu.VMEM((EPT,), jnp.int32),
            pltpu.VMEM((EPT,), jnp.int32),
            pltpu.VMEM((NCH3, CH), jnp.int32),
            pltpu.VMEM((EPT,), jnp.float32),
            pltpu.VMEM((CH, 128), jnp.float32),
            pltpu.VMEM((128, 128), jnp.float32),
            pltpu.VMEM_SHARED((ACC + 8, 128), jnp.float32),
            pltpu.SemaphoreType.DMA,
        ],
    )
    return f(row_pad, col_pad, coef, ff)


BM = 1024
MB = NPAD // BM


def _mm_body(a0, a1, w0, w1, f0, f1, s_ref, b_ref, out_ref):
    s = s_ref[...]
    acc = jnp.zeros(out_ref.shape, jnp.float32) + b_ref[0]
    for a, w, f in ((a0, w0, f0), (a1, w1, f1)):
        m = a[...] + s * f[...]
        acc = acc + jnp.dot(m, w[0, 0], preferred_element_type=jnp.float32,
                            precision=lax.Precision.HIGHEST)
    out_ref[...] = acc


def _mm_call(aggr, lwb, ff, s_col, lbb, split_out):
    aspec = [pl.BlockSpec((BM, 128), lambda m, c, h=h: (h * MB + m, 0))
             for h in range(2)]
    wspec = [pl.BlockSpec((1, 1, 128, 128), lambda m, c, h=h: (h, c, 0, 0))
             for h in range(2)]
    fspec = [pl.BlockSpec((BM, 128), lambda m, c, h=h: (h * MB + m, 0))
             for h in range(2)]
    in_specs = aspec + wspec + fspec + [
        pl.BlockSpec((BM, 1), lambda m, c: (m, 0)),
        pl.BlockSpec((1, 1, 128), lambda m, c: (c, 0, 0)),
    ]
    if split_out:
        out_spec = pl.BlockSpec((BM, 128), lambda m, c: (c * MB + m, 0))
        out_shape = jax.ShapeDtypeStruct((2 * NPAD, 128), jnp.float32)
    else:
        out_spec = pl.BlockSpec((BM, 128), lambda m, c: (m, c))
        out_shape = jax.ShapeDtypeStruct((NPAD, F), jnp.float32)
    return pl.pallas_call(
        _mm_body,
        grid=(MB, 2),
        in_specs=in_specs,
        out_specs=out_spec,
        out_shape=out_shape,
    )(aggr, aggr, lwb, lwb, ff, ff, s_col, lbb)


def _pair_body(i0_hbm, i1_hbm, coors_hbm, f2_hbm, pv_hbm, out_hbm,
               i0_v, i1_v, pv_v, f0_v, f1_v, c0_v, c1_v, out_v, sem):
    wid = _wid()
    base = wid * PPW
    pltpu.sync_copy(i0_hbm.at[pl.ds(base, PPW)], i0_v)
    pltpu.sync_copy(i1_hbm.at[pl.ds(base, PPW)], i1_v)
    pltpu.sync_copy(pv_hbm, pv_v)
    pvv = pv_v[...]
    fw0 = pvv[4]
    fw1 = pvv[5]
    fb = pvv[6]
    lane = _iota16()

    def chunk(j, _):
        d0 = pltpu.async_copy(f2_hbm.at[i0_v.at[pl.ds(j * CH, CH)]], f0_v, sem)
        d1 = pltpu.async_copy(f2_hbm.at[i1_v.at[pl.ds(j * CH, CH)]], f1_v, sem)
        d2 = pltpu.async_copy(coors_hbm.at[i0_v.at[pl.ds(j * CH, CH)]], c0_v, sem)
        d3 = pltpu.async_copy(coors_hbm.at[i1_v.at[pl.ds(j * CH, CH)]], c1_v, sem)
        d0.wait()
        d1.wait()
        d2.wait()
        d3.wait()

        def grp(g, _):
            pidx = g * 16 + lane
            accd = jnp.zeros((16,), jnp.float32)
            for d in range(F):
                dv = jnp.full((16,), d, jnp.int32)
                accd = accd + (plsc.load_gather(f0_v, [pidx, dv])
                               * plsc.load_gather(f1_v, [pidx, dv]))
            accp = jnp.zeros((16,), jnp.float32)
            for d in range(PD):
                dv = jnp.full((16,), d, jnp.int32)
                dd = (plsc.load_gather(c0_v, [pidx, dv])
                      - plsc.load_gather(c1_v, [pidx, dv]))
                accp = accp + dd * dd
            out_v[pl.ds(j * CH + g * 16, 16)] = fw0 * accd + fw1 * accp + fb
            return 0
        lax.fori_loop(0, CH // 16, grp, 0)
        return 0
    lax.fori_loop(0, NCH5, chunk, 0)
    pltpu.sync_copy(out_v, out_hbm.at[pl.ds(base, PPW)])


def _pair_call(i0, i1, coors, f2, pv):
    f = pl.kernel(
        _pair_body,
        out_type=jax.ShapeDtypeStruct((NPAIR,), jnp.float32),
        mesh=_mesh(),
        compiler_params=_SC_PARAMS,
        scratch_types=[
            pltpu.VMEM((PPW,), jnp.int32),
            pltpu.VMEM((PPW,), jnp.int32),
            pltpu.VMEM((16,), jnp.float32),
            pltpu.VMEM((CH, F), jnp.float32),
            pltpu.VMEM((CH, F), jnp.float32),
            pltpu.VMEM((CH, PD), jnp.float32),
            pltpu.VMEM((CH, PD), jnp.float32),
            pltpu.VMEM((PPW,), jnp.float32),
            pltpu.SemaphoreType.DMA,
        ],
    )
    return f(i0, i1, coors, f2, pv)


def kernel(x, edge_index, idx, c1_ew1, c1_eb1, c1_ew2, c1_eb2, c1_lw, c1_lb,
           c2_ew1, c2_eb1, c2_ew2, c2_eb2, c2_lw, c2_lb, fc_w, fc_b):
    row = edge_index[0]
    col = edge_index[1]
    pad = jnp.zeros((EPAD - E,), jnp.int32)
    row_pad = jnp.concatenate([row, pad])
    col_pad = jnp.concatenate([col, pad])
    deg_part = _deg_call(col_pad).reshape(NW, N)
    coors, ff, dis_r, s1_r, s2_r, pv_r = _prep_call(
        x, deg_part,
        c1_ew1, c1_eb1.reshape(1, 32), c1_ew2.reshape(1, 32),
        c1_eb2.reshape(1, 1),
        c2_ew1, c2_eb1.reshape(1, 32), c2_ew2.reshape(1, 32),
        c2_eb2.reshape(1, 1),
        fc_w.reshape(1, 2), fc_b.reshape(1, 1),
    )
    dis = dis_r.reshape(N // 16, 16)
    pv = pv_r.reshape(16)
    s1 = jnp.pad(s1_r.reshape(N, 1), ((0, NPAD - N), (0, 0)))
    s2 = jnp.pad(s2_r.reshape(N, 1), ((0, NPAD - N), (0, 0)))

    coef1, coef2 = _coef_call(row_pad, col_pad, coors, dis, pv)

    lw1b = c1_lw.reshape(2, 128, 2, 128).transpose(0, 2, 1, 3)
    lb1b = c1_lb.reshape(2, 1, 128)
    lw2b = c2_lw.reshape(2, 128, 2, 128).transpose(0, 2, 1, 3)
    lb2b = c2_lb.reshape(2, 1, 128)

    aggr1 = _spmm_call(row_pad, col_pad, coef1, ff)
    ff2 = _mm_call(aggr1, lw1b, ff, s1, lb1b, split_out=True)
    aggr2 = _spmm_call(row_pad, col_pad, coef2, ff2)
    feats2 = _mm_call(aggr2, lw2b, ff2, s2, lb2b, split_out=False)

    out = _pair_call(idx[0], idx[1], coors, feats2, pv)
    return out.reshape(NPAIR, 1)


# speedup vs baseline: 2.8116x; 1.9671x over previous
import functools

import jax
import jax.numpy as jnp
from jax import lax
from jax.experimental import pallas as pl
from jax.experimental.pallas import tpu as pltpu
from jax.experimental.pallas import tpu_sc as plsc

N = 10000
PD = 128
F = 256
E = 160000
EPAD = 163840
NW = 32
EPW = EPAD // NW
EPT = EPAD // 16
NPAIR = 65536
PPW = NPAIR // NW
CH = 128
NCH1 = EPW // CH
NCH3 = EPT // CH
NCH5 = PPW // CH
NPAD = 10240
NROW_T = NPAD // 16


def _mesh():
    return plsc.VectorSubcoreMesh(core_axis_name="c", subcore_axis_name="s")


_SC_PARAMS = pltpu.CompilerParams(needs_layout_passes=False)


def _wid():
    return lax.axis_index("c") * 16 + lax.axis_index("s")


def _iota16():
    return lax.broadcasted_iota(jnp.int32, (16,), 0)


def _deg_body(col_hbm, deg_out, col_v, deg_v):
    wid = _wid()
    base = wid * EPW
    pltpu.sync_copy(col_hbm.at[pl.ds(base, EPW)], col_v)

    lane = _iota16()

    def zero2(i, _):
        plsc.store_scatter(deg_v, [jnp.full((16,), i, jnp.int32), lane],
                           jnp.zeros((16,), jnp.float32))
        return 0
    lax.fori_loop(0, N // 16, zero2, 0)

    def upd(i, _):
        cv = col_v[pl.ds(i * 16, 16)]
        gid = base + i * 16 + lane
        val = jnp.where(gid < E, jnp.float32(1.0), jnp.float32(0.0))
        plsc.addupdate_scatter(deg_v, [cv >> 4, cv & 15], val)
        return 0
    lax.fori_loop(0, EPW // 16, upd, 0)
    pltpu.sync_copy(deg_v, deg_out.at[wid])


def _deg_call(col_pad):
    f = pl.kernel(
        _deg_body,
        out_type=jax.ShapeDtypeStruct((NW, N // 16, 16), jnp.float32),
        mesh=_mesh(),
        compiler_params=_SC_PARAMS,
        scratch_types=[
            pltpu.VMEM((EPW,), jnp.int32),
            pltpu.VMEM((N // 16, 16), jnp.float32),
        ],
    )
    return f(col_pad)


def _prep_body(x_ref, dp_ref, e1w1, e1b1, e1w2, e1b2, e2w1, e2b1, e2w2, e2b2,
               fcw, fcb, coors_ref, ff_ref, dis_ref, s1_ref, s2_ref, pv_ref):
    xv = x_ref[...]
    coors_ref[...] = xv[:, :PD]
    ff_ref[0:N, :] = xv[:, PD:PD + 128]
    ff_ref[NPAD:NPAD + N, :] = xv[:, PD + 128:]
    deg = jnp.sum(dp_ref[...], axis=0, keepdims=True) + 1.0
    dis_ref[...] = lax.rsqrt(deg)
    a1 = jnp.sum(e1w1[...] * e1w2[...])
    c1 = jnp.sum(e1b1[...] * e1w2[...]) + e1b2[0, 0]
    a2 = jnp.sum(e2w1[...] * e2w2[...])
    c2 = jnp.sum(e2b1[...] * e2w2[...]) + e2b2[0, 0]
    s1_ref[...] = jax.nn.sigmoid(c1) / deg
    s2_ref[...] = jax.nn.sigmoid(c2) / deg
    k = lax.broadcasted_iota(jnp.int32, (1, 16), 1)
    pv = jnp.where(k == 0, a1, 0.0)
    pv = jnp.where(k == 1, c1, pv)
    pv = jnp.where(k == 2, a2, pv)
    pv = jnp.where(k == 3, c2, pv)
    pv = jnp.where(k == 4, fcw[0, 0], pv)
    pv = jnp.where(k == 5, fcw[0, 1], pv)
    pv = jnp.where(k == 6, fcb[0, 0], pv)
    pv_ref[...] = pv


def _prep_call(x, dp, e1w1, e1b1, e1w2, e1b2, e2w1, e2b1, e2w2, e2b2, fcw, fcb):
    return pl.pallas_call(
        _prep_body,
        out_shape=[
            jax.ShapeDtypeStruct((N, PD), jnp.float32),
            jax.ShapeDtypeStruct((2 * NPAD, 128), jnp.float32),
            jax.ShapeDtypeStruct((1, N), jnp.float32),
            jax.ShapeDtypeStruct((1, N), jnp.float32),
            jax.ShapeDtypeStruct((1, N), jnp.float32),
            jax.ShapeDtypeStruct((1, 16), jnp.float32),
        ],
    )(x, dp, e1w1, e1b1, e1w2, e1b2, e2w1, e2b1, e2w2, e2b2, fcw, fcb)


def _coef_body(row_hbm, col_hbm, coors_hbm, dis_hbm, pv_hbm,
               coef1_hbm, coef2_hbm,
               row_v, col_v, dis_v, pv_v, cR, cC, c1_v, c2_v, sem):
    wid = _wid()
    base = wid * EPW
    pltpu.sync_copy(dis_hbm, dis_v)
    pltpu.sync_copy(pv_hbm, pv_v)
    pltpu.sync_copy(row_hbm.at[pl.ds(base, EPW)], row_v)
    pltpu.sync_copy(col_hbm.at[pl.ds(base, EPW)], col_v)
    pvv = pv_v[...]
    a1 = pvv[0]
    cc1 = pvv[1]
    a2 = pvv[2]
    cc2 = pvv[3]
    lane = _iota16()

    def chunk(j, _):
        cpR = pltpu.async_copy(coors_hbm.at[row_v.at[pl.ds(j * CH, CH)]], cR, sem)
        cpC = pltpu.async_copy(coors_hbm.at[col_v.at[pl.ds(j * CH, CH)]], cC, sem)
        cpR.wait()
        cpC.wait()

        def grp(g, _):
            rel = jnp.zeros((16,), jnp.float32)
            for t in range(16):
                e = g * 16 + t
                acc = [jnp.zeros((16,), jnp.float32) for _ in range(4)]
                for v in range(8):
                    df = cR[e, pl.ds(v * 16, 16)] - cC[e, pl.ds(v * 16, 16)]
                    acc[v % 4] = acc[v % 4] + df * df
                rel_e = jnp.sum((acc[0] + acc[1]) + (acc[2] + acc[3]))
                rel = jnp.where(lane == t, rel_e, rel)
            rlo = row_v[pl.ds(j * CH + g * 16, 16)]
            clo = col_v[pl.ds(j * CH + g * 16, 16)]
            enorm = (plsc.load_gather(dis_v, [rlo >> 4, rlo & 15])
                     * plsc.load_gather(dis_v, [clo >> 4, clo & 15]))
            gid = base + j * CH + g * 16 + lane
            valid = gid < E
            w1 = 1.0 / (1.0 + jnp.exp(-(a1 * rel + cc1)))
            w2 = 1.0 / (1.0 + jnp.exp(-(a2 * rel + cc2)))
            c1_v[pl.ds(g * 16, 16)] = jnp.where(valid, enorm * w1, 0.0)
            c2_v[pl.ds(g * 16, 16)] = jnp.where(valid, enorm * w2, 0.0)
            return 0
        lax.fori_loop(0, CH // 16, grp, 0)
        pltpu.sync_copy(c1_v, coef1_hbm.at[pl.ds(base + j * CH, CH)])
        pltpu.sync_copy(c2_v, coef2_hbm.at[pl.ds(base + j * CH, CH)])
        return 0
    lax.fori_loop(0, NCH1, chunk, 0)


def _coef_call(row_pad, col_pad, coors, dis, pv):
    f = pl.kernel(
        _coef_body,
        out_type=(
            jax.ShapeDtypeStruct((EPAD,), jnp.float32),
            jax.ShapeDtypeStruct((EPAD,), jnp.float32),
        ),
        mesh=_mesh(),
        compiler_params=_SC_PARAMS,
        scratch_types=[
            pltpu.VMEM((EPW,), jnp.int32),
            pltpu.VMEM((EPW,), jnp.int32),
            pltpu.VMEM((N // 16, 16), jnp.float32),
            pltpu.VMEM((16,), jnp.float32),
            pltpu.VMEM((CH, PD), jnp.float32),
            pltpu.VMEM((CH, PD), jnp.float32),
            pltpu.VMEM((CH,), jnp.float32),
            pltpu.VMEM((CH,), jnp.float32),
            pltpu.SemaphoreType.DMA,
        ],
    )
    return f(row_pad, col_pad, coors, dis, pv)


ACC = 6144


def _spmm_body(row_hbm, col_hbm, coef_hbm, ff_hbm, aggr_hbm,
               row_v, col_v, colp_v, coef_v, msg_v, zero_v, agg_sh, sem):
    cidx = lax.axis_index("c")
    sidx = lax.axis_index("s")
    tbase = sidx * EPT
    pltpu.sync_copy(row_hbm.at[pl.ds(tbase, EPT)], row_v)
    pltpu.sync_copy(col_hbm.at[pl.ds(tbase, EPT)], col_v)
    pltpu.sync_copy(coef_hbm.at[pl.ds(tbase, EPT)], coef_v)

    off = cidx * NPAD

    def adj(i, _):
        row_v[pl.ds(i * 16, 16)] = row_v[pl.ds(i * 16, 16)] + off
        return 0
    lax.fori_loop(0, EPT // 16, adj, 0)

    def zbuf(i, _):
        zero_v[i // 8, pl.ds((i % 8) * 16, 16)] = jnp.zeros((16,), jnp.float32)
        return 0
    lax.fori_loop(0, (128 * 128) // 16, zbuf, 0)

    for pp in range(2):
        pbase = pp * ACC
        prows = ACC if pp == 0 else NPAD - ACC
        rows_t = prows // 16

        def redir(i, _):
            r = i // 8
            v = i % 8
            cv = col_v[pl.ds(i * 16, 16)]
            lv = cv - pbase
            ok = (lv >= 0) & (lv < prows)
            colp_v[r, pl.ds(v * 16, 16)] = jnp.where(ok, lv, ACC)
            return 0
        lax.fori_loop(0, EPT // 16, redir, 0)

        for k in range(rows_t // 128):
            pltpu.sync_copy(zero_v,
                            agg_sh.at[pl.ds(sidx * rows_t + k * 128, 128)])
        plsc.subcore_barrier()

        def chunk(j, _):
            pltpu.async_copy(ff_hbm.at[row_v.at[pl.ds(j * CH, CH)]], msg_v,
                             sem).wait()

            def scale(e2, _):
                for t in range(4):
                    e = e2 * 4 + t
                    csv = plsc.load_gather(
                        coef_v, [jnp.full((16,), j * CH + e, jnp.int32)])
                    for v in range(8):
                        msg_v[e, pl.ds(v * 16, 16)] = (
                            msg_v[e, pl.ds(v * 16, 16)] * csv)
                return 0
            lax.fori_loop(0, CH // 4, scale, 0)
            pltpu.sync_copy(msg_v, agg_sh.at[colp_v.at[j]], add=True)
            return 0
        lax.fori_loop(0, NCH3, chunk, 0)
        plsc.subcore_barrier()
        pltpu.sync_copy(
            agg_sh.at[pl.ds(sidx * rows_t, rows_t)],
            aggr_hbm.at[pl.ds(cidx * NPAD + pbase + sidx * rows_t, rows_t)])
        if pp == 0:
            plsc.subcore_barrier()


def _spmm_call(row_pad, col_pad, coef, ff):
    f = pl.kernel(
        _spmm_body,
        out_type=jax.ShapeDtypeStruct((2 * NPAD, 128), jnp.float32),
        mesh=_mesh(),
        compiler_params=_SC_PARAMS,
        scratch_types=[
            pltpu.VMEM((EPT,), jnp.int32),
            pltpu.VMEM((EPT,), jnp.int32),
            pltpu.VMEM((NCH3, CH), jnp.int32),
            pltpu.VMEM((EPT,), jnp.float32),
            pltpu.VMEM((CH, 128), jnp.float32),
            pltpu.VMEM((128, 128), jnp.float32),
            pltpu.VMEM_SHARED((ACC + 8, 128), jnp.float32),
            pltpu.SemaphoreType.DMA,
        ],
    )
    return f(row_pad, col_pad, coef, ff)


BM = 1024
MB = NPAD // BM


def _mm_body(a0, a1, w0, w1, f0, f1, s_ref, b_ref, out_ref):
    s = s_ref[...]
    acc = jnp.zeros(out_ref.shape, jnp.float32) + b_ref[0]
    for a, w, f in ((a0, w0, f0), (a1, w1, f1)):
        m = a[...] + s * f[...]
        acc = acc + jnp.dot(m, w[0, 0], preferred_element_type=jnp.float32,
                            precision=lax.Precision.HIGHEST)
    out_ref[...] = acc


def _mm_call(aggr, lwb, ff, s_col, lbb, split_out):
    aspec = [pl.BlockSpec((BM, 128), lambda m, c, h=h: (h * MB + m, 0))
             for h in range(2)]
    wspec = [pl.BlockSpec((1, 1, 128, 128), lambda m, c, h=h: (h, c, 0, 0))
             for h in range(2)]
    fspec = [pl.BlockSpec((BM, 128), lambda m, c, h=h: (h * MB + m, 0))
             for h in range(2)]
    in_specs = aspec + wspec + fspec + [
        pl.BlockSpec((BM, 1), lambda m, c: (m, 0)),
        pl.BlockSpec((1, 1, 128), lambda m, c: (c, 0, 0)),
    ]
    if split_out:
        out_spec = pl.BlockSpec((BM, 128), lambda m, c: (c * MB + m, 0))
        out_shape = jax.ShapeDtypeStruct((2 * NPAD, 128), jnp.float32)
    else:
        out_spec = pl.BlockSpec((BM, 128), lambda m, c: (m, c))
        out_shape = jax.ShapeDtypeStruct((NPAD, F), jnp.float32)
    return pl.pallas_call(
        _mm_body,
        grid=(MB, 2),
        in_specs=in_specs,
        out_specs=out_spec,
        out_shape=out_shape,
    )(aggr, aggr, lwb, lwb, ff, ff, s_col, lbb)


CH5 = 64
NCH5 = PPW // CH5


def _pair_body(i0_hbm, i1_hbm, coors_hbm, f2_hbm, pv_hbm, out_hbm,
               i0_v, i1_v, pv_v, f0_v, f1_v, c0_v, c1_v, out_v, sem):
    wid = _wid()
    base = wid * PPW
    pltpu.sync_copy(i0_hbm.at[pl.ds(base, PPW)], i0_v)
    pltpu.sync_copy(i1_hbm.at[pl.ds(base, PPW)], i1_v)
    pltpu.sync_copy(pv_hbm, pv_v)
    pvv = pv_v[...]
    fw0 = pvv[4]
    fw1 = pvv[5]
    fb = pvv[6]
    lane = _iota16()

    def chunk(j, _):
        d0 = pltpu.async_copy(f2_hbm.at[i0_v.at[pl.ds(j * CH5, CH5)]], f0_v, sem)
        d1 = pltpu.async_copy(f2_hbm.at[i1_v.at[pl.ds(j * CH5, CH5)]], f1_v, sem)
        d2 = pltpu.async_copy(coors_hbm.at[i0_v.at[pl.ds(j * CH5, CH5)]], c0_v, sem)
        d3 = pltpu.async_copy(coors_hbm.at[i1_v.at[pl.ds(j * CH5, CH5)]], c1_v, sem)
        d0.wait()
        d1.wait()
        d2.wait()
        d3.wait()

        def grp(g, _):
            res = jnp.zeros((16,), jnp.float32)
            for t in range(16):
                p = g * 16 + t
                acc = [jnp.zeros((16,), jnp.float32) for _ in range(4)]
                for v in range(16):
                    acc[v % 4] = acc[v % 4] + (f0_v[p, pl.ds(v * 16, 16)]
                                               * f1_v[p, pl.ds(v * 16, 16)])
                pacc = [jnp.zeros((16,), jnp.float32) for _ in range(4)]
                for v in range(8):
                    dd = c0_v[p, pl.ds(v * 16, 16)] - c1_v[p, pl.ds(v * 16, 16)]
                    pacc[v % 4] = pacc[v % 4] + dd * dd
                dot_e = jnp.sum((acc[0] + acc[1]) + (acc[2] + acc[3]))
                pe_e = jnp.sum((pacc[0] + pacc[1]) + (pacc[2] + pacc[3]))
                res = jnp.where(lane == t, fw0 * dot_e + fw1 * pe_e + fb, res)
            out_v[pl.ds(j * CH5 + g * 16, 16)] = res
            return 0
        lax.fori_loop(0, CH5 // 16, grp, 0)
        return 0
    lax.fori_loop(0, NCH5, chunk, 0)
    pltpu.sync_copy(out_v, out_hbm.at[pl.ds(base, PPW)])


def _pair_call(i0, i1, coors, f2, pv):
    f = pl.kernel(
        _pair_body,
        out_type=jax.ShapeDtypeStruct((NPAIR,), jnp.float32),
        mesh=_mesh(),
        compiler_params=_SC_PARAMS,
        scratch_types=[
            pltpu.VMEM((PPW,), jnp.int32),
            pltpu.VMEM((PPW,), jnp.int32),
            pltpu.VMEM((16,), jnp.float32),
            pltpu.VMEM((CH5, F), jnp.float32),
            pltpu.VMEM((CH5, F), jnp.float32),
            pltpu.VMEM((CH5, PD), jnp.float32),
            pltpu.VMEM((CH5, PD), jnp.float32),
            pltpu.VMEM((PPW,), jnp.float32),
            pltpu.SemaphoreType.DMA,
        ],
    )
    return f(i0, i1, coors, f2, pv)


def kernel(x, edge_index, idx, c1_ew1, c1_eb1, c1_ew2, c1_eb2, c1_lw, c1_lb,
           c2_ew1, c2_eb1, c2_ew2, c2_eb2, c2_lw, c2_lb, fc_w, fc_b):
    row = edge_index[0]
    col = edge_index[1]
    pad = jnp.zeros((EPAD - E,), jnp.int32)
    row_pad = jnp.concatenate([row, pad])
    col_pad = jnp.concatenate([col, pad])
    deg_part = _deg_call(col_pad).reshape(NW, N)
    coors, ff, dis_r, s1_r, s2_r, pv_r = _prep_call(
        x, deg_part,
        c1_ew1, c1_eb1.reshape(1, 32), c1_ew2.reshape(1, 32),
        c1_eb2.reshape(1, 1),
        c2_ew1, c2_eb1.reshape(1, 32), c2_ew2.reshape(1, 32),
        c2_eb2.reshape(1, 1),
        fc_w.reshape(1, 2), fc_b.reshape(1, 1),
    )
    dis = dis_r.reshape(N // 16, 16)
    pv = pv_r.reshape(16)
    s1 = jnp.pad(s1_r.reshape(N, 1), ((0, NPAD - N), (0, 0)))
    s2 = jnp.pad(s2_r.reshape(N, 1), ((0, NPAD - N), (0, 0)))

    coef1, coef2 = _coef_call(row_pad, col_pad, coors, dis, pv)

    lw1b = c1_lw.reshape(2, 128, 2, 128).transpose(0, 2, 1, 3)
    lb1b = c1_lb.reshape(2, 1, 128)
    lw2b = c2_lw.reshape(2, 128, 2, 128).transpose(0, 2, 1, 3)
    lb2b = c2_lb.reshape(2, 1, 128)

    aggr1 = _spmm_call(row_pad, col_pad, coef1, ff)
    ff2 = _mm_call(aggr1, lw1b, ff, s1, lb1b, split_out=True)
    aggr2 = _spmm_call(row_pad, col_pad, coef2, ff2)
    feats2 = _mm_call(aggr2, lw2b, ff2, s2, lb2b, split_out=False)

    out = _pair_call(idx[0], idx[1], coors, feats2, pv)
    return out.reshape(NPAIR, 1)
